# baseline (speedup 1.0000x reference)
import functools

import jax
import jax.numpy as jnp
from jax import lax
from jax.experimental import pallas as pl
from jax.experimental.pallas import tpu as pltpu
from jax.experimental.pallas import tpu_sc as plsc

N = 10000
E = 320000
D = 128
VOCAB = 100
NUM_LAYERS = 4
AVG_DEG = float(E) / float(N)
PW = 16

NC, NS = 2, 16
NW = NC * NS
EPW = E // NW
CH = 80
NCHUNK = EPW // CH
NPAD = 10240
NPT = NPAD // NS
EPC = E // NC

@functools.lru_cache(maxsize=1)
def _sc_mesh():
  return plsc.VectorSubcoreMesh(
      core_axis_name="c", subcore_axis_name="s", num_cores=NC, num_subcores=NS)


TW = 2 * D
UL = D + PW


def _gather_body(A, B, src_hbm, dst_hbm, t_out, idxs_v, idxd_v, g1_v, g2_v,
                 sem):
  cid = lax.axis_index("c")
  sid = lax.axis_index("s")
  wid = sid * NC + cid
  base = wid * EPW
  pltpu.sync_copy(src_hbm.at[pl.ds(base, EPW)], idxs_v)
  pltpu.sync_copy(dst_hbm.at[pl.ds(base, EPW)], idxd_v)

  def chunk(i, carry):
    c1 = pltpu.async_copy(A.at[idxs_v.at[pl.ds(i * CH, CH)]], g1_v, sem)
    c2 = pltpu.async_copy(B.at[idxd_v.at[pl.ds(i * CH, CH)]], g2_v, sem)
    c1.wait(); c2.wait()

    def row(r, carry2):
      for j in range(UL // 16):
        sl = pl.ds(j * 16, 16)
        g1_v[r, sl] = g1_v[r, sl] + g2_v[r, sl]
      return carry2
    lax.fori_loop(0, CH, row, 0)

    pltpu.sync_copy(g1_v, t_out.at[pl.ds(base + i * CH, CH)])
    return carry
  lax.fori_loop(0, NCHUNK, chunk, 0)


@functools.partial(jax.jit, static_argnums=())
def _gather_call(A, B, src, dst):
  return pl.kernel(
      _gather_body,
      out_type=jax.ShapeDtypeStruct((E, TW), jnp.float32),
      mesh=_sc_mesh(),
      scratch_types=[
          pltpu.VMEM((EPW,), jnp.int32),
          pltpu.VMEM((EPW,), jnp.int32),
          pltpu.VMEM((CH, TW), jnp.float32),
          pltpu.VMEM((CH, TW), jnp.float32),
          pltpu.SemaphoreType.DMA,
      ],
  )(A, B, src, dst)


def _scatter_body(m_hbm, cmf_hbm, dst_hbm, zrow, aggp, caccp,
                  idx_v, m_v, c10_v, acc_sh):
  cid = lax.axis_index("c")
  sid = lax.axis_index("s")
  base = cid * EPC + sid * EPW

  pltpu.sync_copy(zrow, acc_sh.at[pl.ds(sid * NPT, NPT)])
  plsc.subcore_barrier()

  def chunk_m(i, carry):
    off = base + i * CH
    pltpu.sync_copy(dst_hbm.at[pl.ds(off, CH)], idx_v)
    pltpu.sync_copy(m_hbm.at[pl.ds(off, CH)], m_v)
    pltpu.sync_copy(m_v, acc_sh.at[idx_v], add=True)
    return carry
  lax.fori_loop(0, NCHUNK, chunk_m, 0)

  plsc.subcore_barrier()
  pltpu.sync_copy(acc_sh.at[pl.ds(sid * NPT, NPT)],
                  aggp.at[cid, pl.ds(sid * NPT, NPT), :])
  plsc.subcore_barrier()

  pltpu.sync_copy(zrow, acc_sh.at[pl.ds(sid * NPT, NPT)])

  def zrv(r, carry):
    for j in range(D // 16):
      m_v[r, pl.ds(j * 16, 16)] = jnp.zeros((16,), jnp.float32)
    return carry
  lax.fori_loop(0, CH, zrv, 0)
  plsc.subcore_barrier()

  def chunk_c(i, carry):
    off = base + i * CH
    pltpu.sync_copy(dst_hbm.at[pl.ds(off, CH)], idx_v)
    pltpu.sync_copy(cmf_hbm.at[pl.ds(off * PW, CH * PW)], c10_v)

    def urow(r, carry2):
      m_v[r, pl.ds(0, PW)] = c10_v[pl.ds(r * PW, PW)]
      return carry2
    lax.fori_loop(0, CH, urow, 0)
    pltpu.sync_copy(m_v, acc_sh.at[idx_v], add=True)
    return carry
  lax.fori_loop(0, NCHUNK, chunk_c, 0)

  plsc.subcore_barrier()
  pltpu.sync_copy(acc_sh.at[pl.ds(sid * NPT, NPT)],
                  caccp.at[cid, pl.ds(sid * NPT, NPT), :])


@functools.partial(jax.jit, static_argnums=())
def _scatter_call(m, cmf, dst, zrow):
  return pl.kernel(
      _scatter_body,
      out_type=(jax.ShapeDtypeStruct((NC, NPAD, D), jnp.float32),
                jax.ShapeDtypeStruct((NC, NPAD, D), jnp.float32)),
      mesh=_sc_mesh(),
      scratch_types=[
          pltpu.VMEM((CH,), jnp.int32),
          pltpu.VMEM((CH, D), jnp.float32),
          pltpu.VMEM((CH * PW,), jnp.float32),
          pltpu.VMEM_SHARED((NPAD, D), jnp.float32),
      ],
  )(m, cmf, dst, zrow)


BE = 3200


def _edge_tc(t_ref, relf_ref, w2_ref, b2_ref, w1c_ref, cw_ref, cb_ref,
             m_ref, cm_ref):
  t = t_ref[...]
  relf = relf_ref[...]
  d2 = jnp.sum(relf * relf, axis=1, keepdims=True)
  pre = t + d2 * w1c_ref[...]
  u = pre * jax.nn.sigmoid(pre)
  z = jnp.dot(u, w2_ref[...], preferred_element_type=jnp.float32) + b2_ref[...]
  m = z * jax.nn.sigmoid(z)
  s = jnp.sum(m * cw_ref[...], axis=1, keepdims=True) + cb_ref[...]
  m_ref[...] = m
  cm_ref[...] = relf[:, :PW] * s


@functools.partial(jax.jit, static_argnums=())
def _edge_call(t_ext, w2, b2, w1c, cw, cb):
  full = lambda a: pl.BlockSpec(a.shape, lambda i: (0,) * a.ndim)
  return pl.pallas_call(
      _edge_tc,
      grid=(E // BE,),
      in_specs=[
          pl.BlockSpec((BE, D), lambda i: (i, 0)),
          pl.BlockSpec((BE, D), lambda i: (i, 1)),
          full(w2), full(b2), full(w1c), full(cw), full(cb),
      ],
      out_specs=[
          pl.BlockSpec((BE, D), lambda i: (i, 0)),
          pl.BlockSpec((BE, PW), lambda i: (i, 0)),
      ],
      out_shape=(jax.ShapeDtypeStruct((E, D), jnp.float32),
                 jax.ShapeDtypeStruct((E, PW), jnp.float32)),
  )(t_ext, t_ext, w2, b2, w1c, cw, cb)


BN = 2000


def _node_tc(h_ref, a0_ref, a1_ref, c0_ref, c1_ref, pos_ref,
             nw1a_ref, nw1b_ref, nb1_ref, nw2_ref, nb2_ref,
             ew1a_ref, ew1b_ref, eb1_ref,
             hn_ref, posn_ref, an_ref, bn_ref):
  h = h_ref[...]
  agg = a0_ref[...] + a1_ref[...]
  upd = (jnp.dot(h, nw1a_ref[...], preferred_element_type=jnp.float32)
         + jnp.dot(agg, nw1b_ref[...], preferred_element_type=jnp.float32)
         + nb1_ref[...])
  upd = upd * jax.nn.sigmoid(upd)
  hn = h + jnp.dot(upd, nw2_ref[...], preferred_element_type=jnp.float32) + nb2_ref[...]
  hn_ref[...] = hn
  posn = pos_ref[...] + (c0_ref[...] + c1_ref[...]) * (1.0 / AVG_DEG)
  posn_ref[...] = posn
  zpad = jnp.zeros((posn.shape[0], TW - D - PW), jnp.float32)
  an_ref[...] = jnp.concatenate(
      [jnp.dot(hn, ew1a_ref[...], preferred_element_type=jnp.float32),
       posn, zpad], axis=1)
  bn_ref[...] = jnp.concatenate(
      [jnp.dot(hn, ew1b_ref[...], preferred_element_type=jnp.float32)
       + eb1_ref[...], -posn, zpad], axis=1)


@functools.partial(jax.jit, static_argnums=())
def _node_call(h, a0, a1, c0, c1, pos16, nw1a, nw1b, nb1, nw2, nb2,
               ew1a, ew1b, eb1):
  full = lambda a: pl.BlockSpec(a.shape, lambda i: (0,) * a.ndim)
  return pl.pallas_call(
      _node_tc,
      grid=(N // BN,),
      in_specs=[
          pl.BlockSpec((BN, D), lambda i: (i, 0)),
          pl.BlockSpec((BN, D), lambda i: (i, 0)),
          pl.BlockSpec((BN, D), lambda i: (i, 0)),
          pl.BlockSpec((BN, PW), lambda i: (i, 0)),
          pl.BlockSpec((BN, PW), lambda i: (i, 0)),
          pl.BlockSpec((BN, PW), lambda i: (i, 0)),
          full(nw1a), full(nw1b), full(nb1), full(nw2), full(nb2),
          full(ew1a), full(ew1b), full(eb1),
      ],
      out_specs=[
          pl.BlockSpec((BN, D), lambda i: (i, 0)),
          pl.BlockSpec((BN, PW), lambda i: (i, 0)),
          pl.BlockSpec((BN, TW), lambda i: (i, 0)),
          pl.BlockSpec((BN, TW), lambda i: (i, 0)),
      ],
      out_shape=(jax.ShapeDtypeStruct((N, D), jnp.float32),
                 jax.ShapeDtypeStruct((N, PW), jnp.float32),
                 jax.ShapeDtypeStruct((N, TW), jnp.float32),
                 jax.ShapeDtypeStruct((N, TW), jnp.float32)),
  )(h, a0, a1, c0, c1, pos16, nw1a, nw1b, nb1, nw2, nb2, ew1a, ew1b, eb1)


def _init_tc(x_ref, pos_ref, emb_ref, ew1a_ref, ew1b_ref, eb1_ref,
             h_ref, a_ref, b_ref):
  xi = x_ref[...]
  lanes = lax.broadcasted_iota(jnp.int32, (BN, D), 1)
  oh = (lanes == xi).astype(jnp.float32)
  h = jnp.dot(oh, emb_ref[...], preferred_element_type=jnp.float32)
  h_ref[...] = h
  pos = pos_ref[...]
  zpad = jnp.zeros((BN, TW - D - PW), jnp.float32)
  a_ref[...] = jnp.concatenate(
      [jnp.dot(h, ew1a_ref[...], preferred_element_type=jnp.float32),
       pos, zpad], axis=1)
  b_ref[...] = jnp.concatenate(
      [jnp.dot(h, ew1b_ref[...], preferred_element_type=jnp.float32)
       + eb1_ref[...], -pos, zpad], axis=1)


@functools.partial(jax.jit, static_argnums=())
def _init_call(x, pos16, emb_pad, ew1a, ew1b, eb1):
  full = lambda a: pl.BlockSpec(a.shape, lambda i: (0,) * a.ndim)
  return pl.pallas_call(
      _init_tc,
      grid=(N // BN,),
      in_specs=[
          pl.BlockSpec((BN, 1), lambda i: (i, 0)),
          pl.BlockSpec((BN, PW), lambda i: (i, 0)),
          full(emb_pad), full(ew1a), full(ew1b), full(eb1),
      ],
      out_specs=[
          pl.BlockSpec((BN, D), lambda i: (i, 0)),
          pl.BlockSpec((BN, TW), lambda i: (i, 0)),
          pl.BlockSpec((BN, TW), lambda i: (i, 0)),
      ],
      out_shape=(jax.ShapeDtypeStruct((N, D), jnp.float32),
                 jax.ShapeDtypeStruct((N, TW), jnp.float32),
                 jax.ShapeDtypeStruct((N, TW), jnp.float32)),
  )(x, pos16, emb_pad, ew1a, ew1b, eb1)


def _pool_tc(h_ref, pw1_ref, pb1_ref, pw2_ref, pb2_ref, out_ref, acc):
  i = pl.program_id(0)

  @pl.when(i == 0)
  def _zero():
    acc[...] = jnp.zeros_like(acc)

  acc[0:1, :] += jnp.sum(h_ref[...], axis=0, keepdims=True)

  @pl.when(i == pl.num_programs(0) - 1)
  def _final():
    patch = acc[0:1, :] * (1.0 / N)
    hid = jnp.maximum(
        jnp.dot(patch, pw1_ref[...], preferred_element_type=jnp.float32)
        + pb1_ref[...], 0.0)
    out_ref[...] = jnp.dot(hid, pw2_ref[...], preferred_element_type=jnp.float32) + pb2_ref[...]


@functools.partial(jax.jit, static_argnums=())
def _pool_call(h, pw1, pb1, pw2, pb2):
  full = lambda a: pl.BlockSpec(a.shape, lambda i: (0,) * a.ndim)
  return pl.pallas_call(
      _pool_tc,
      grid=(N // BN,),
      in_specs=[
          pl.BlockSpec((BN, D), lambda i: (i, 0)),
          full(pw1), full(pb1), full(pw2), full(pb2),
      ],
      out_specs=pl.BlockSpec((1, D), lambda i: (0, 0)),
      out_shape=jax.ShapeDtypeStruct((1, D), jnp.float32),
      scratch_shapes=[pltpu.VMEM((8, D), jnp.float32)],
  )(h, pw1, pb1, pw2, pb2)


def kernel(x, pos, edge_index, params):
  src = edge_index[0]
  dst = edge_index[1]
  pos16 = jnp.pad(pos, ((0, 0), (0, PW - 3)))

  emb_pad = jnp.pad(params['emb'], ((0, D - VOCAB), (0, 0)))
  zrow = jnp.zeros((NPT, D), jnp.float32)
  zw = jnp.zeros((D, D), jnp.float32)
  zb = jnp.zeros((1, D), jnp.float32)

  lps = params['layers']
  ew1a = [lp['edge_W1'][:D, :] for lp in lps] + [zw]
  ew1b = [lp['edge_W1'][D:2 * D, :] for lp in lps] + [zw]
  eb1 = [lp['edge_b1'].reshape(1, D) for lp in lps] + [zb]
  w1c = [lp['edge_W1'][2 * D:2 * D + 1, :] for lp in lps]
  w2 = [lp['edge_W2'] for lp in lps]
  b2 = [lp['edge_b2'].reshape(1, D) for lp in lps]
  cw = [lp['coord_W'].reshape(1, D) for lp in lps]
  cb = [lp['coord_b'].reshape(1, 1) for lp in lps]
  nw1a = [lp['node_W1'][:D, :] for lp in lps]
  nw1b = [lp['node_W1'][D:, :] for lp in lps]
  nb1 = [lp['node_b1'].reshape(1, D) for lp in lps]
  nw2 = [lp['node_W2'] for lp in lps]
  nb2 = [lp['node_b2'].reshape(1, D) for lp in lps]

  h, A, B = _init_call(x, pos16, emb_pad, ew1a[0], ew1b[0], eb1[0])
  for l in range(NUM_LAYERS):
    t_ext = _gather_call(A, B, src, dst)
    m, cm = _edge_call(t_ext, w2[l], b2[l], w1c[l], cw[l], cb[l])
    aggp, caccp = _scatter_call(m, cm.reshape(E * PW), dst, zrow)
    h, pos16, A, B = _node_call(
        h, aggp[0, :N], aggp[1, :N], caccp[0, :N, :PW], caccp[1, :N, :PW],
        pos16,
        nw1a[l], nw1b[l], nb1[l], nw2[l], nb2[l],
        ew1a[l + 1], ew1b[l + 1], eb1[l + 1])

  out = _pool_call(h, params['pool_W1'], params['pool_b1'].reshape(1, D),
                   params['pool_W2'], params['pool_b2'].reshape(1, D))
  return out.reshape(D)


# speedup vs baseline: 1.0685x; 1.0685x over previous
import functools

import jax
import jax.numpy as jnp
from jax import lax
from jax.experimental import pallas as pl
from jax.experimental.pallas import tpu as pltpu
from jax.experimental.pallas import tpu_sc as plsc

N = 10000
E = 320000
D = 128
VOCAB = 100
NUM_LAYERS = 4
AVG_DEG = float(E) / float(N)
PW = 16

NC, NS = 2, 16
NW = NC * NS
EPW = E // NW
CH = 80
NCHUNK = EPW // CH
NPAD = 10240
NPT = NPAD // NS
EPC = E // NC

@functools.lru_cache(maxsize=1)
def _sc_mesh():
  return plsc.VectorSubcoreMesh(
      core_axis_name="c", subcore_axis_name="s", num_cores=NC, num_subcores=NS)


TW = 2 * D
UL = D + PW


def _gather_body(A, B, src_hbm, dst_hbm, t_out, idxs_v, idxd_v, g1_v, g2_v,
                 sem):
  cid = lax.axis_index("c")
  sid = lax.axis_index("s")
  wid = sid * NC + cid
  base = wid * EPW
  pltpu.sync_copy(src_hbm.at[pl.ds(base, EPW)], idxs_v)
  pltpu.sync_copy(dst_hbm.at[pl.ds(base, EPW)], idxd_v)

  def chunk(i, carry):
    c1 = pltpu.async_copy(A.at[idxs_v.at[pl.ds(i * CH, CH)]], g1_v, sem)
    c2 = pltpu.async_copy(B.at[idxd_v.at[pl.ds(i * CH, CH)]], g2_v, sem)
    c1.wait(); c2.wait()

    def row(r, carry2):
      for j in range(UL // 16):
        sl = pl.ds(j * 16, 16)
        g1_v[r, sl] = g1_v[r, sl] + g2_v[r, sl]
      return carry2
    lax.fori_loop(0, CH, row, 0)

    pltpu.sync_copy(g1_v, t_out.at[pl.ds(base + i * CH, CH)])
    return carry
  lax.fori_loop(0, NCHUNK, chunk, 0)


@functools.partial(jax.jit, static_argnums=())
def _gather_call(A, B, src, dst):
  return pl.kernel(
      _gather_body,
      out_type=jax.ShapeDtypeStruct((E, TW), jnp.float32),
      mesh=_sc_mesh(),
      scratch_types=[
          pltpu.VMEM((EPW,), jnp.int32),
          pltpu.VMEM((EPW,), jnp.int32),
          pltpu.VMEM((CH, TW), jnp.float32),
          pltpu.VMEM((CH, TW), jnp.float32),
          pltpu.SemaphoreType.DMA,
      ],
  )(A, B, src, dst)


def _scatter_body(m_hbm, cm_hbm, dst_hbm, zrow, aggp, caccp,
                  idx_v, m_v, acc_sh):
  cid = lax.axis_index("c")
  sid = lax.axis_index("s")
  base = cid * EPC + sid * EPW

  def one_pass(src_hbm, outp):
    pltpu.sync_copy(zrow, acc_sh.at[pl.ds(sid * NPT, NPT)])
    plsc.subcore_barrier()

    def chunk(i, carry):
      off = base + i * CH
      pltpu.sync_copy(dst_hbm.at[pl.ds(off, CH)], idx_v)
      pltpu.sync_copy(src_hbm.at[pl.ds(off, CH)], m_v)
      pltpu.sync_copy(m_v, acc_sh.at[idx_v], add=True)
      return carry
    lax.fori_loop(0, NCHUNK, chunk, 0)

    plsc.subcore_barrier()
    pltpu.sync_copy(acc_sh.at[pl.ds(sid * NPT, NPT)],
                    outp.at[cid, pl.ds(sid * NPT, NPT), :])
    plsc.subcore_barrier()

  one_pass(m_hbm, aggp)
  one_pass(cm_hbm, caccp)


@functools.partial(jax.jit, static_argnums=())
def _scatter_call(m, cm, dst, zrow):
  return pl.kernel(
      _scatter_body,
      out_type=(jax.ShapeDtypeStruct((NC, NPAD, D), jnp.float32),
                jax.ShapeDtypeStruct((NC, NPAD, D), jnp.float32)),
      mesh=_sc_mesh(),
      scratch_types=[
          pltpu.VMEM((CH,), jnp.int32),
          pltpu.VMEM((CH, D), jnp.float32),
          pltpu.VMEM_SHARED((NPAD, D), jnp.float32),
      ],
  )(m, cm, dst, zrow)


BE = 3200


def _edge_tc(t_ref, relf_ref, w2_ref, b2_ref, w1c_ref, cw_ref, cb_ref,
             m_ref, cm_ref):
  t = t_ref[...]
  relf = relf_ref[...]
  d2 = jnp.sum(relf * relf, axis=1, keepdims=True)
  pre = t + d2 * w1c_ref[...]
  u = pre * jax.nn.sigmoid(pre)
  z = jnp.dot(u, w2_ref[...], preferred_element_type=jnp.float32) + b2_ref[...]
  m = z * jax.nn.sigmoid(z)
  s = jnp.sum(m * cw_ref[...], axis=1, keepdims=True) + cb_ref[...]
  m_ref[...] = m
  cm_ref[...] = relf * s


@functools.partial(jax.jit, static_argnums=())
def _edge_call(t_ext, w2, b2, w1c, cw, cb):
  full = lambda a: pl.BlockSpec(a.shape, lambda i: (0,) * a.ndim)
  return pl.pallas_call(
      _edge_tc,
      grid=(E // BE,),
      in_specs=[
          pl.BlockSpec((BE, D), lambda i: (i, 0)),
          pl.BlockSpec((BE, D), lambda i: (i, 1)),
          full(w2), full(b2), full(w1c), full(cw), full(cb),
      ],
      out_specs=[
          pl.BlockSpec((BE, D), lambda i: (i, 0)),
          pl.BlockSpec((BE, D), lambda i: (i, 0)),
      ],
      out_shape=(jax.ShapeDtypeStruct((E, D), jnp.float32),
                 jax.ShapeDtypeStruct((E, D), jnp.float32)),
  )(t_ext, t_ext, w2, b2, w1c, cw, cb)


BN = 2000


def _node_tc(h_ref, a0_ref, a1_ref, c0_ref, c1_ref, pos_ref,
             nw1a_ref, nw1b_ref, nb1_ref, nw2_ref, nb2_ref,
             ew1a_ref, ew1b_ref, eb1_ref,
             hn_ref, posn_ref, an_ref, bn_ref):
  h = h_ref[...]
  agg = a0_ref[...] + a1_ref[...]
  upd = (jnp.dot(h, nw1a_ref[...], preferred_element_type=jnp.float32)
         + jnp.dot(agg, nw1b_ref[...], preferred_element_type=jnp.float32)
         + nb1_ref[...])
  upd = upd * jax.nn.sigmoid(upd)
  hn = h + jnp.dot(upd, nw2_ref[...], preferred_element_type=jnp.float32) + nb2_ref[...]
  hn_ref[...] = hn
  posn = pos_ref[...] + (c0_ref[...] + c1_ref[...]) * (1.0 / AVG_DEG)
  posn_ref[...] = posn
  zpad = jnp.zeros((posn.shape[0], TW - D - PW), jnp.float32)
  an_ref[...] = jnp.concatenate(
      [jnp.dot(hn, ew1a_ref[...], preferred_element_type=jnp.float32),
       posn, zpad], axis=1)
  bn_ref[...] = jnp.concatenate(
      [jnp.dot(hn, ew1b_ref[...], preferred_element_type=jnp.float32)
       + eb1_ref[...], -posn, zpad], axis=1)


@functools.partial(jax.jit, static_argnums=())
def _node_call(h, a0, a1, c0, c1, pos16, nw1a, nw1b, nb1, nw2, nb2,
               ew1a, ew1b, eb1):
  full = lambda a: pl.BlockSpec(a.shape, lambda i: (0,) * a.ndim)
  return pl.pallas_call(
      _node_tc,
      grid=(N // BN,),
      in_specs=[
          pl.BlockSpec((BN, D), lambda i: (i, 0)),
          pl.BlockSpec((BN, D), lambda i: (i, 0)),
          pl.BlockSpec((BN, D), lambda i: (i, 0)),
          pl.BlockSpec((BN, PW), lambda i: (i, 0)),
          pl.BlockSpec((BN, PW), lambda i: (i, 0)),
          pl.BlockSpec((BN, PW), lambda i: (i, 0)),
          full(nw1a), full(nw1b), full(nb1), full(nw2), full(nb2),
          full(ew1a), full(ew1b), full(eb1),
      ],
      out_specs=[
          pl.BlockSpec((BN, D), lambda i: (i, 0)),
          pl.BlockSpec((BN, PW), lambda i: (i, 0)),
          pl.BlockSpec((BN, TW), lambda i: (i, 0)),
          pl.BlockSpec((BN, TW), lambda i: (i, 0)),
      ],
      out_shape=(jax.ShapeDtypeStruct((N, D), jnp.float32),
                 jax.ShapeDtypeStruct((N, PW), jnp.float32),
                 jax.ShapeDtypeStruct((N, TW), jnp.float32),
                 jax.ShapeDtypeStruct((N, TW), jnp.float32)),
  )(h, a0, a1, c0, c1, pos16, nw1a, nw1b, nb1, nw2, nb2, ew1a, ew1b, eb1)


def _init_tc(x_ref, pos_ref, emb_ref, ew1a_ref, ew1b_ref, eb1_ref,
             h_ref, a_ref, b_ref):
  xi = x_ref[...]
  lanes = lax.broadcasted_iota(jnp.int32, (BN, D), 1)
  oh = (lanes == xi).astype(jnp.float32)
  h = jnp.dot(oh, emb_ref[...], preferred_element_type=jnp.float32)
  h_ref[...] = h
  pos = pos_ref[...]
  zpad = jnp.zeros((BN, TW - D - PW), jnp.float32)
  a_ref[...] = jnp.concatenate(
      [jnp.dot(h, ew1a_ref[...], preferred_element_type=jnp.float32),
       pos, zpad], axis=1)
  b_ref[...] = jnp.concatenate(
      [jnp.dot(h, ew1b_ref[...], preferred_element_type=jnp.float32)
       + eb1_ref[...], -pos, zpad], axis=1)


@functools.partial(jax.jit, static_argnums=())
def _init_call(x, pos16, emb_pad, ew1a, ew1b, eb1):
  full = lambda a: pl.BlockSpec(a.shape, lambda i: (0,) * a.ndim)
  return pl.pallas_call(
      _init_tc,
      grid=(N // BN,),
      in_specs=[
          pl.BlockSpec((BN, 1), lambda i: (i, 0)),
          pl.BlockSpec((BN, PW), lambda i: (i, 0)),
          full(emb_pad), full(ew1a), full(ew1b), full(eb1),
      ],
      out_specs=[
          pl.BlockSpec((BN, D), lambda i: (i, 0)),
          pl.BlockSpec((BN, TW), lambda i: (i, 0)),
          pl.BlockSpec((BN, TW), lambda i: (i, 0)),
      ],
      out_shape=(jax.ShapeDtypeStruct((N, D), jnp.float32),
                 jax.ShapeDtypeStruct((N, TW), jnp.float32),
                 jax.ShapeDtypeStruct((N, TW), jnp.float32)),
  )(x, pos16, emb_pad, ew1a, ew1b, eb1)


def _pool_tc(h_ref, pw1_ref, pb1_ref, pw2_ref, pb2_ref, out_ref, acc):
  i = pl.program_id(0)

  @pl.when(i == 0)
  def _zero():
    acc[...] = jnp.zeros_like(acc)

  acc[0:1, :] += jnp.sum(h_ref[...], axis=0, keepdims=True)

  @pl.when(i == pl.num_programs(0) - 1)
  def _final():
    patch = acc[0:1, :] * (1.0 / N)
    hid = jnp.maximum(
        jnp.dot(patch, pw1_ref[...], preferred_element_type=jnp.float32)
        + pb1_ref[...], 0.0)
    out_ref[...] = jnp.dot(hid, pw2_ref[...], preferred_element_type=jnp.float32) + pb2_ref[...]


@functools.partial(jax.jit, static_argnums=())
def _pool_call(h, pw1, pb1, pw2, pb2):
  full = lambda a: pl.BlockSpec(a.shape, lambda i: (0,) * a.ndim)
  return pl.pallas_call(
      _pool_tc,
      grid=(N // BN,),
      in_specs=[
          pl.BlockSpec((BN, D), lambda i: (i, 0)),
          full(pw1), full(pb1), full(pw2), full(pb2),
      ],
      out_specs=pl.BlockSpec((1, D), lambda i: (0, 0)),
      out_shape=jax.ShapeDtypeStruct((1, D), jnp.float32),
      scratch_shapes=[pltpu.VMEM((8, D), jnp.float32)],
  )(h, pw1, pb1, pw2, pb2)


def kernel(x, pos, edge_index, params):
  src = edge_index[0]
  dst = edge_index[1]
  pos16 = jnp.pad(pos, ((0, 0), (0, PW - 3)))

  emb_pad = jnp.pad(params['emb'], ((0, D - VOCAB), (0, 0)))
  zrow = jnp.zeros((NPT, D), jnp.float32)
  zw = jnp.zeros((D, D), jnp.float32)
  zb = jnp.zeros((1, D), jnp.float32)

  lps = params['layers']
  ew1a = [lp['edge_W1'][:D, :] for lp in lps] + [zw]
  ew1b = [lp['edge_W1'][D:2 * D, :] for lp in lps] + [zw]
  eb1 = [lp['edge_b1'].reshape(1, D) for lp in lps] + [zb]
  w1c = [lp['edge_W1'][2 * D:2 * D + 1, :] for lp in lps]
  w2 = [lp['edge_W2'] for lp in lps]
  b2 = [lp['edge_b2'].reshape(1, D) for lp in lps]
  cw = [lp['coord_W'].reshape(1, D) for lp in lps]
  cb = [lp['coord_b'].reshape(1, 1) for lp in lps]
  nw1a = [lp['node_W1'][:D, :] for lp in lps]
  nw1b = [lp['node_W1'][D:, :] for lp in lps]
  nb1 = [lp['node_b1'].reshape(1, D) for lp in lps]
  nw2 = [lp['node_W2'] for lp in lps]
  nb2 = [lp['node_b2'].reshape(1, D) for lp in lps]

  h, A, B = _init_call(x, pos16, emb_pad, ew1a[0], ew1b[0], eb1[0])
  for l in range(NUM_LAYERS):
    t_ext = _gather_call(A, B, src, dst)
    m, cm = _edge_call(t_ext, w2[l], b2[l], w1c[l], cw[l], cb[l])
    aggp, caccp = _scatter_call(m, cm, dst, zrow)
    h, pos16, A, B = _node_call(
        h, aggp[0, :N], aggp[1, :N], caccp[0, :N, :PW], caccp[1, :N, :PW],
        pos16,
        nw1a[l], nw1b[l], nb1[l], nw2[l], nb2[l],
        ew1a[l + 1], ew1b[l + 1], eb1[l + 1])

  out = _pool_call(h, params['pool_W1'], params['pool_b1'].reshape(1, D),
                   params['pool_W2'], params['pool_b2'].reshape(1, D))
  return out.reshape(D)
